# baseline (speedup 1.0000x reference)
import functools

import jax
import jax.numpy as jnp
from jax.experimental import pallas as pl
from jax.experimental.pallas import tpu as pltpu


def _se_step(x_ref, w1_ref, b1_ref, w2_ref, b2_ref, o_ref, *, inv_hw):
    x = x_ref[...]
    m = jnp.sum(x, axis=-1, dtype=jnp.float32) * inv_hw
    h = jax.lax.dot_general(m, w1_ref[...], (((1,), (1,)), ((), ())),
                            preferred_element_type=jnp.float32)
    h = jnp.maximum(h + b1_ref[...], 0.0)
    z = jax.lax.dot_general(h, w2_ref[...], (((1,), (1,)), ((), ())),
                            preferred_element_type=jnp.float32)
    s = jax.nn.sigmoid(z + b2_ref[...])
    o_ref[...] = (x * s[:, :, None].astype(x.dtype)).astype(o_ref.dtype)


def kernel(x, w1, b1, w2, b2):
    B, C, H, W = x.shape
    Cr = w1.shape[0]
    HW = H * W

    x_flat = x.astype(jnp.bfloat16).reshape(B, C, HW)
    b1r = b1.astype(jnp.float32).reshape(1, Cr)
    b2r = b2.astype(jnp.float32).reshape(1, C)
    w1f = w1.astype(jnp.float32)
    w2f = w2.astype(jnp.float32)

    out_flat = pl.pallas_call(
        functools.partial(_se_step, inv_hw=1.0 / HW),
        out_shape=jax.ShapeDtypeStruct((B, C, HW), jnp.bfloat16),
        grid=(B,),
        in_specs=[
            pl.BlockSpec((1, C, HW), lambda b: (b, 0, 0)),
            pl.BlockSpec((Cr, C), lambda b: (0, 0)),
            pl.BlockSpec((1, Cr), lambda b: (0, 0)),
            pl.BlockSpec((C, Cr), lambda b: (0, 0)),
            pl.BlockSpec((1, C), lambda b: (0, 0)),
        ],
        out_specs=pl.BlockSpec((1, C, HW), lambda b: (b, 0, 0)),
        compiler_params=pltpu.CompilerParams(
            dimension_semantics=("parallel",),
            vmem_limit_bytes=44 << 20,
        ),
        cost_estimate=pl.CostEstimate(
            flops=int(2 * B * C * HW + 4 * B * C * Cr),
            transcendentals=int(B * C),
            bytes_accessed=int(2 * B * C * HW * 2),
        ),
    )(x_flat, w1f, b1r, w2f, b2r)

    return out_flat.reshape(B, C, H, W).astype(x.dtype)


# speedup vs baseline: 1.0121x; 1.0121x over previous
import functools

import jax
import jax.numpy as jnp
from jax.experimental import pallas as pl
from jax.experimental.pallas import tpu as pltpu


def _se_step(x_ref, w1_ref, b1_ref, w2_ref, b2_ref, o_ref, *, inv_hw):
    x = x_ref[...]
    m = jnp.sum(x, axis=-1, dtype=jnp.float32) * inv_hw
    h = jax.lax.dot_general(m, w1_ref[...], (((1,), (1,)), ((), ())),
                            preferred_element_type=jnp.float32)
    h = jnp.maximum(h + b1_ref[...], 0.0)
    z = jax.lax.dot_general(h, w2_ref[...], (((1,), (1,)), ((), ())),
                            preferred_element_type=jnp.float32)
    s = jax.nn.sigmoid(z + b2_ref[...])
    o_ref[...] = (x * s[:, :, None].astype(x.dtype)).astype(o_ref.dtype)


def kernel(x, w1, b1, w2, b2):
    B, C, H, W = x.shape
    Cr = w1.shape[0]
    HW = H * W

    x_flat = x.reshape(B, C, HW).astype(jnp.bfloat16)
    b1r = b1.astype(jnp.float32).reshape(1, Cr)
    b2r = b2.astype(jnp.float32).reshape(1, C)
    w1f = w1.astype(jnp.float32)
    w2f = w2.astype(jnp.float32)

    out_flat = pl.pallas_call(
        functools.partial(_se_step, inv_hw=1.0 / HW),
        out_shape=jax.ShapeDtypeStruct((B, C, HW), jnp.bfloat16),
        grid=(B // 2,),
        in_specs=[
            pl.BlockSpec((2, C, HW), lambda b: (b, 0, 0)),
            pl.BlockSpec((Cr, C), lambda b: (0, 0)),
            pl.BlockSpec((1, Cr), lambda b: (0, 0)),
            pl.BlockSpec((C, Cr), lambda b: (0, 0)),
            pl.BlockSpec((1, C), lambda b: (0, 0)),
        ],
        out_specs=pl.BlockSpec((2, C, HW), lambda b: (b, 0, 0)),
        compiler_params=pltpu.CompilerParams(
            dimension_semantics=("parallel",),
            vmem_limit_bytes=44 << 20,
        ),
        cost_estimate=pl.CostEstimate(
            flops=int(2 * B * C * HW + 4 * B * C * Cr),
            transcendentals=int(B * C),
            bytes_accessed=int(2 * B * C * HW * 2),
        ),
    )(x_flat, w1f, b1r, w2f, b2r)

    return out_flat.astype(x.dtype).reshape(B, C, H, W)


# speedup vs baseline: 1.0181x; 1.0059x over previous
import functools

import jax
import jax.numpy as jnp
from jax.experimental import pallas as pl
from jax.experimental.pallas import tpu as pltpu


def _se_step(x_ref, w1_ref, b1_ref, w2_ref, b2_ref, o_ref, *, inv_hw):
    x = x_ref[...]
    m = jnp.sum(x, axis=-1, dtype=jnp.float32) * inv_hw
    h = jax.lax.dot_general(m, w1_ref[...], (((1,), (1,)), ((), ())),
                            preferred_element_type=jnp.float32)
    h = jnp.maximum(h + b1_ref[...], 0.0)
    z = jax.lax.dot_general(h, w2_ref[...], (((1,), (1,)), ((), ())),
                            preferred_element_type=jnp.float32)
    s = jax.nn.sigmoid(z + b2_ref[...])
    o_ref[...] = (x * s[:, :, None].astype(x.dtype)).astype(o_ref.dtype)


def kernel(x, w1, b1, w2, b2):
    B, C, H, W = x.shape
    Cr = w1.shape[0]
    HW = H * W

    x_flat = x.reshape(B, C, HW).astype(jnp.bfloat16)
    b1r = b1.astype(jnp.float32).reshape(1, Cr)
    b2r = b2.astype(jnp.float32).reshape(1, C)
    w1f = w1.astype(jnp.float32)
    w2f = w2.astype(jnp.float32)

    out_flat = pl.pallas_call(
        functools.partial(_se_step, inv_hw=1.0 / HW),
        out_shape=jax.ShapeDtypeStruct((B, C, HW), jnp.bfloat16),
        grid=(B // 4,),
        in_specs=[
            pl.BlockSpec((4, C, HW), lambda b: (b, 0, 0)),
            pl.BlockSpec((Cr, C), lambda b: (0, 0)),
            pl.BlockSpec((1, Cr), lambda b: (0, 0)),
            pl.BlockSpec((C, Cr), lambda b: (0, 0)),
            pl.BlockSpec((1, C), lambda b: (0, 0)),
        ],
        out_specs=pl.BlockSpec((4, C, HW), lambda b: (b, 0, 0)),
        compiler_params=pltpu.CompilerParams(
            dimension_semantics=("parallel",),
            vmem_limit_bytes=56 << 20,
        ),
        cost_estimate=pl.CostEstimate(
            flops=int(2 * B * C * HW + 4 * B * C * Cr),
            transcendentals=int(B * C),
            bytes_accessed=int(2 * B * C * HW * 2),
        ),
    )(x_flat, w1f, b1r, w2f, b2r)

    return out_flat.astype(x.dtype).reshape(B, C, H, W)
